# baseline (speedup 1.0000x reference)
import functools

import jax
import jax.numpy as jnp
from jax import lax
from jax.experimental import pallas as pl
from jax.experimental.pallas import tpu as pltpu
from jax.experimental.pallas import tpu_sc as plsc

BATCH = 1024
SEQ = 200
D = 64
N = BATCH * SEQ
NC, NS = 2, 16
NW = NC * NS
R = N // NW
C = 128
G = R // C

_mesh = plsc.VectorSubcoreMesh(core_axis_name="c", subcore_axis_name="s")


@functools.partial(
    pl.kernel,
    mesh=_mesh,
    compiler_params=pltpu.CompilerParams(use_tc_tiling_on_sc=False),
    out_type=jax.ShapeDtypeStruct((N, D), jnp.float32),
    scratch_types=[
        pltpu.VMEM((R,), jnp.int32),
        pltpu.VMEM((C, D), jnp.float32),
        pltpu.VMEM((C, D), jnp.float32),
        pltpu.SemaphoreType.DMA,
        pltpu.SemaphoreType.DMA,
    ],
)
def _emb_add(x_hbm, idx_hbm, w_hbm, out_hbm, idx_v, x_v, w_v, sem_x, sem_w):
    wid = lax.axis_index("s") * NC + lax.axis_index("c")
    base = wid * R
    pltpu.sync_copy(idx_hbm.at[pl.ds(base, R)], idx_v)

    def chunk(g, carry):
        r0 = base + g * C
        cpx = pltpu.make_async_copy(x_hbm.at[pl.ds(r0, C)], x_v, sem_x)
        cpx.start()
        cpx.wait()
        cpw = pltpu.async_copy(
            w_hbm.at[idx_v.at[pl.ds(g * C, C)]], x_v, sem_w, add=True)
        cpw.wait()
        pltpu.sync_copy(x_v, out_hbm.at[pl.ds(r0, C)])
        return carry

    lax.fori_loop(0, G, chunk, 0)


def kernel(x, pos_seq, position_weight):
    x2 = x.reshape(N, D)
    idx = pos_seq.reshape(N)
    out = _emb_add(x2, idx, position_weight)
    return out.reshape(BATCH, SEQ, D)


# speedup vs baseline: 1.0209x; 1.0209x over previous
import functools

import jax
import jax.numpy as jnp
from jax import lax
from jax.experimental import pallas as pl
from jax.experimental.pallas import tpu as pltpu
from jax.experimental.pallas import tpu_sc as plsc

BATCH = 1024
SEQ = 200
D = 64
N = BATCH * SEQ
NC, NS = 2, 16
NW = NC * NS
R = N // NW
C = 256
G = R // C
SUB = C // 128
NBUF = 4
STEPS = G + 2
OUTER = (STEPS + NBUF - 1) // NBUF

_mesh = plsc.VectorSubcoreMesh(core_axis_name="c", subcore_axis_name="s")

_scratch = (
    [pltpu.VMEM((R,), jnp.int32)]
    + [pltpu.VMEM((C, D), jnp.float32) for _ in range(NBUF)]
    + [pltpu.SemaphoreType.DMA for _ in range(3 * NBUF)]
)


@functools.partial(
    pl.kernel,
    mesh=_mesh,
    compiler_params=pltpu.CompilerParams(use_tc_tiling_on_sc=False),
    out_type=jax.ShapeDtypeStruct((N, D), jnp.float32),
    scratch_types=_scratch,
)
def _emb_add(x_hbm, idx_hbm, w_hbm, out_hbm, idx_v, *rest):
    bufs = list(rest[:NBUF])
    sxs = list(rest[NBUF:2 * NBUF])
    sws = list(rest[2 * NBUF:3 * NBUF])
    sos = list(rest[3 * NBUF:4 * NBUF])

    wid = lax.axis_index("s") * NC + lax.axis_index("c")
    base = wid * R
    pltpu.sync_copy(idx_hbm.at[pl.ds(base, R)], idx_v)

    def start_x(i, b):
        pltpu.async_copy(x_hbm.at[pl.ds(base + i * C, C)], bufs[b], sxs[b])

    def wait_x(b):
        pltpu.make_async_copy(x_hbm.at[pl.ds(0, C)], bufs[b], sxs[b]).wait()

    def start_w(i, b):
        for s in range(SUB):
            pltpu.async_copy(
                w_hbm.at[idx_v.at[pl.ds(i * C + s * 128, 128)]],
                bufs[b].at[pl.ds(s * 128, 128)],
                sws[b],
                add=True,
            )

    def wait_w(b):
        for s in range(SUB):
            pltpu.make_async_copy(
                w_hbm.at[idx_v.at[pl.ds(s * 128, 128)]],
                bufs[b].at[pl.ds(s * 128, 128)],
                sws[b],
            ).wait()

    def start_o(i, b):
        pltpu.async_copy(bufs[b], out_hbm.at[pl.ds(base + i * C, C)], sos[b])

    def wait_o(b):
        pltpu.make_async_copy(x_hbm.at[pl.ds(0, C)], bufs[b], sos[b]).wait()

    def outer(i2, carry):
        for u in range(NBUF):
            i = i2 * NBUF + u

            @pl.when(i < G)
            def _():
                @pl.when(i >= NBUF)
                def _():
                    wait_o(u)
                start_x(i, u)

            ib = i - 1
            bb = (u - 1) % NBUF

            @pl.when(jnp.logical_and(ib >= 0, ib < G))
            def _():
                wait_x(bb)
                start_w(ib, bb)

            ic = i - 2
            bc = (u - 2) % NBUF

            @pl.when(jnp.logical_and(ic >= 0, ic < G))
            def _():
                wait_w(bc)
                start_o(ic, bc)

        return carry

    lax.fori_loop(0, OUTER, outer, 0)

    for b in range(NBUF):
        wait_o(b)


def kernel(x, pos_seq, position_weight):
    x2 = x.reshape(N, D)
    idx = pos_seq.reshape(N)
    out = _emb_add(x2, idx, position_weight)
    return out.reshape(BATCH, SEQ, D)


# speedup vs baseline: 1.0214x; 1.0005x over previous
import functools

import jax
import jax.numpy as jnp
from jax import lax
from jax.experimental import pallas as pl
from jax.experimental.pallas import tpu as pltpu
from jax.experimental.pallas import tpu_sc as plsc

BATCH = 1024
SEQ = 200
D = 64
NC, NS = 2, 16
NW = NC * NS
BPW = BATCH // NW
G = BPW
NBUF = 4
STEPS = G + 2
OUTER = (STEPS + NBUF - 1) // NBUF

_mesh = plsc.VectorSubcoreMesh(core_axis_name="c", subcore_axis_name="s")

_scratch = (
    [pltpu.VMEM((BPW, SEQ), jnp.int32)]
    + [pltpu.VMEM((SEQ, D), jnp.float32) for _ in range(NBUF)]
    + [pltpu.SemaphoreType.DMA for _ in range(3 * NBUF)]
)


@functools.partial(
    pl.kernel,
    mesh=_mesh,
    compiler_params=pltpu.CompilerParams(use_tc_tiling_on_sc=False),
    out_type=jax.ShapeDtypeStruct((BATCH, SEQ, D), jnp.float32),
    scratch_types=_scratch,
)
def _emb_add(x_hbm, idx_hbm, w_hbm, out_hbm, idx_v, *rest):
    bufs = list(rest[:NBUF])
    sxs = list(rest[NBUF:2 * NBUF])
    sws = list(rest[2 * NBUF:3 * NBUF])
    sos = list(rest[3 * NBUF:4 * NBUF])

    wid = lax.axis_index("s") * NC + lax.axis_index("c")
    base = wid * BPW
    pltpu.sync_copy(idx_hbm.at[pl.ds(base, BPW), :], idx_v)

    SPLITS = ((0, 128), (128, SEQ - 128))

    def start_x(i, b):
        pltpu.async_copy(x_hbm.at[base + i], bufs[b], sxs[b])

    def wait_x(b):
        pltpu.make_async_copy(x_hbm.at[0], bufs[b], sxs[b]).wait()

    def start_w(i, b):
        for off, ln in SPLITS:
            pltpu.async_copy(
                w_hbm.at[idx_v.at[i, pl.ds(off, ln)]],
                bufs[b].at[pl.ds(off, ln)],
                sws[b],
                add=True,
            )

    def wait_w(b):
        for off, ln in SPLITS:
            pltpu.make_async_copy(
                w_hbm.at[idx_v.at[0, pl.ds(off, ln)]],
                bufs[b].at[pl.ds(off, ln)],
                sws[b],
            ).wait()

    def start_o(i, b):
        pltpu.async_copy(bufs[b], out_hbm.at[base + i], sos[b])

    def wait_o(b):
        pltpu.make_async_copy(x_hbm.at[0], bufs[b], sos[b]).wait()

    def outer(i2, carry):
        for u in range(NBUF):
            i = i2 * NBUF + u

            @pl.when(i < G)
            def _():
                @pl.when(i >= NBUF)
                def _():
                    wait_o(u)
                start_x(i, u)

            ib = i - 1
            bb = (u - 1) % NBUF

            @pl.when(jnp.logical_and(ib >= 0, ib < G))
            def _():
                wait_x(bb)
                start_w(ib, bb)

            ic = i - 2
            bc = (u - 2) % NBUF

            @pl.when(jnp.logical_and(ic >= 0, ic < G))
            def _():
                wait_w(bc)
                start_o(ic, bc)

        return carry

    lax.fori_loop(0, OUTER, outer, 0)

    for b in range(NBUF):
        wait_o(b)


def kernel(x, pos_seq, position_weight):
    return _emb_add(x, pos_seq, position_weight)


# speedup vs baseline: 1.3240x; 1.2963x over previous
import functools

import jax
import jax.numpy as jnp
from jax import lax
from jax.experimental import pallas as pl
from jax.experimental.pallas import tpu as pltpu
from jax.experimental.pallas import tpu_sc as plsc

BATCH = 1024
SEQ = 200
D = 64
NC, NS = 2, 16
NW = NC * NS
UNITS = SEQ * (BATCH // 128)
G = UNITS // NW
NBUF = 3
STEPS = G + 1
WROWS = SEQ * D

_mesh = plsc.VectorSubcoreMesh(core_axis_name="c", subcore_axis_name="s")

_scratch = (
    [pltpu.VMEM((WROWS,), jnp.float32)]
    + [pltpu.VMEM((8, 8, 128), jnp.float32) for _ in range(NBUF)]
    + [pltpu.VMEM((128,), jnp.int32) for _ in range(NBUF)]
    + [pltpu.SemaphoreType.DMA for _ in range(2 * NBUF)]
)


@functools.partial(
    pl.kernel,
    mesh=_mesh,
    compiler_params=pltpu.CompilerParams(needs_layout_passes=False),
    out_type=jax.ShapeDtypeStruct((SEQ, 8, 8, 8, 128), jnp.float32),
    scratch_types=_scratch,
)
def _emb_add(x_hbm, p_hbm, w_hbm, out_hbm, wt_v, *rest):
    slabs = list(rest[:NBUF])
    idxs = list(rest[NBUF:2 * NBUF])
    sin = list(rest[2 * NBUF:3 * NBUF])
    sout = list(rest[3 * NBUF:4 * NBUF])

    wid = lax.axis_index("s") * NC + lax.axis_index("c")
    base = wid * G
    pltpu.sync_copy(w_hbm, wt_v)

    def start_in(i, b):
        ug = base + i
        s = ug // 8
        bt = ug % 8
        pltpu.async_copy(x_hbm.at[s, :, bt], slabs[b], sin[b])
        pltpu.async_copy(p_hbm.at[s // 8, bt, s % 8], idxs[b], sin[b])

    def wait_in(b):
        pltpu.make_async_copy(x_hbm.at[0, :, 0], slabs[b], sin[b]).wait()
        pltpu.make_async_copy(p_hbm.at[0, 0, 0], idxs[b], sin[b]).wait()

    def start_o(i, b):
        ug = base + i
        pltpu.async_copy(slabs[b], out_hbm.at[ug // 8, :, ug % 8], sout[b])

    def wait_o(b):
        pltpu.make_async_copy(x_hbm.at[0, :, 0], slabs[b], sout[b]).wait()

    def compute(b):
        def lgroup(l, c):
            sl = pl.ds(l * 16, 16)
            rbase = idxs[b][sl] * D
            for e in range(D):
                v = plsc.load_gather(wt_v, [rbase + e])
                plsc.addupdate(slabs[b].at[e // 8, e % 8, sl], v)
            return c

        lax.fori_loop(0, 8, lgroup, 0)

    def outer(i2, carry):
        for u in range(NBUF):
            i = i2 * NBUF + u

            @pl.when(i < G)
            def _():
                @pl.when(i >= NBUF)
                def _():
                    wait_o(u)
                start_in(i, u)

            ib = i - 1
            bb = (u - 1) % NBUF

            @pl.when(jnp.logical_and(ib >= 0, ib < G))
            def _():
                wait_in(bb)
                compute(bb)
                start_o(ib, bb)

        return carry

    lax.fori_loop(0, STEPS // NBUF, outer, 0)

    for b in range(NBUF):
        wait_o(b)


def kernel(x, pos_seq, position_weight):
    xv = (x.transpose(1, 2, 0)
          .reshape(SEQ, 8, 8, 8, 128)
          .transpose(0, 1, 3, 2, 4))
    pv = (pos_seq.T
          .reshape(SEQ // 8, 8, 8, 128)
          .transpose(0, 2, 1, 3))
    wf = position_weight.reshape(WROWS)
    o5 = _emb_add(xv, pv, wf)
    return (o5.transpose(0, 1, 3, 2, 4)
            .reshape(SEQ, D, BATCH)
            .transpose(2, 0, 1))


# speedup vs baseline: 2.1328x; 1.6109x over previous
import functools

import jax
import jax.numpy as jnp
from jax import lax
from jax.experimental import pallas as pl
from jax.experimental.pallas import tpu as pltpu
from jax.experimental.pallas import tpu_sc as plsc

BATCH = 1024
SEQ = 200
D = 64
NC, NS = 2, 16
NW = NC * NS
UNITS = SEQ * (BATCH // 128)
G = UNITS // NW
NBUF = 3
STEPS = G + 1
WROWS = SEQ * D

_mesh = plsc.VectorSubcoreMesh(core_axis_name="c", subcore_axis_name="s")

_scratch = (
    [pltpu.VMEM((WROWS,), jnp.float32)]
    + [pltpu.VMEM((8, 8, 128), jnp.float32) for _ in range(NBUF)]
    + [pltpu.VMEM((128,), jnp.int32) for _ in range(NBUF)]
    + [pltpu.SemaphoreType.DMA for _ in range(2 * NBUF)]
)


@functools.partial(
    pl.kernel,
    mesh=_mesh,
    compiler_params=pltpu.CompilerParams(needs_layout_passes=False),
    out_type=jax.ShapeDtypeStruct((SEQ, 8, 8, 8, 128), jnp.float32),
    scratch_types=_scratch,
)
def _emb_add(x_hbm, p_hbm, w_hbm, out_hbm, wt_v, *rest):
    slabs = list(rest[:NBUF])
    idxs = list(rest[NBUF:2 * NBUF])
    sin = list(rest[2 * NBUF:3 * NBUF])
    sout = list(rest[3 * NBUF:4 * NBUF])

    wid = lax.axis_index("s") * NC + lax.axis_index("c")
    base = wid * G
    pltpu.sync_copy(w_hbm, wt_v)

    def start_in(i, b):
        ug = base + i
        s = ug // 8
        bt = ug % 8
        pltpu.async_copy(x_hbm.at[s, :, bt], slabs[b], sin[b])
        pltpu.async_copy(p_hbm.at[s // 8, bt, s % 8], idxs[b], sin[b])

    def wait_in(b):
        pltpu.make_async_copy(x_hbm.at[0, :, 0], slabs[b], sin[b]).wait()
        pltpu.make_async_copy(p_hbm.at[0, 0, 0], idxs[b], sin[b]).wait()

    def start_o(i, b):
        ug = base + i
        pltpu.async_copy(slabs[b], out_hbm.at[ug // 8, :, ug % 8], sout[b])

    def wait_o(b):
        pltpu.make_async_copy(x_hbm.at[0, :, 0], slabs[b], sout[b]).wait()

    def compute(b):
        def lgroup(l, c):
            sl = pl.ds(l * 16, 16)
            rbase = idxs[b][sl] * D

            @plsc.parallel_loop(0, D, unroll=8)
            def _(e):
                v = plsc.load_gather(wt_v, [rbase + e])
                plsc.addupdate(slabs[b].at[e // 8, e % 8, sl], v)

            return c

        lax.fori_loop(0, 8, lgroup, 0)

    def outer(i2, carry):
        for u in range(NBUF):
            i = i2 * NBUF + u

            @pl.when(i < G)
            def _():
                @pl.when(i >= NBUF)
                def _():
                    wait_o(u)
                start_in(i, u)

            ib = i - 1
            bb = (u - 1) % NBUF

            @pl.when(jnp.logical_and(ib >= 0, ib < G))
            def _():
                wait_in(bb)
                compute(bb)
                start_o(ib, bb)

        return carry

    lax.fori_loop(0, STEPS // NBUF, outer, 0)

    for b in range(NBUF):
        wait_o(b)


def kernel(x, pos_seq, position_weight):
    xv = (x.transpose(1, 2, 0)
          .reshape(SEQ, 8, 8, 8, 128)
          .transpose(0, 1, 3, 2, 4))
    pv = (pos_seq.T
          .reshape(SEQ // 8, 8, 8, 128)
          .transpose(0, 2, 1, 3))
    wf = position_weight.reshape(WROWS)
    o5 = _emb_add(xv, pv, wf)
    return (o5.transpose(0, 1, 3, 2, 4)
            .reshape(SEQ, D, BATCH)
            .transpose(2, 0, 1))


# speedup vs baseline: 2.1673x; 1.0162x over previous
import functools

import jax
import jax.numpy as jnp
from jax import lax
from jax.experimental import pallas as pl
from jax.experimental.pallas import tpu as pltpu
from jax.experimental.pallas import tpu_sc as plsc

BATCH = 1024
SEQ = 200
D = 64
NC, NS = 2, 16
NW = NC * NS
UNITS = SEQ * (BATCH // 128)
G = UNITS // NW
NBUF = 3
STEPS = G + 1
WROWS = SEQ * D

_mesh = plsc.VectorSubcoreMesh(core_axis_name="c", subcore_axis_name="s")

_scratch = (
    [pltpu.VMEM((WROWS,), jnp.float32)]
    + [pltpu.VMEM((8, 8, 128), jnp.float32) for _ in range(NBUF)]
    + [pltpu.VMEM((128,), jnp.int32) for _ in range(NBUF)]
    + [pltpu.SemaphoreType.DMA for _ in range(2 * NBUF)]
)


@functools.partial(
    pl.kernel,
    mesh=_mesh,
    compiler_params=pltpu.CompilerParams(needs_layout_passes=False),
    out_type=jax.ShapeDtypeStruct((SEQ, 8, 8, 8, 128), jnp.float32),
    scratch_types=_scratch,
)
def _emb_add(x_hbm, p_hbm, w_hbm, out_hbm, wt_v, *rest):
    slabs = list(rest[:NBUF])
    idxs = list(rest[NBUF:2 * NBUF])
    sin = list(rest[2 * NBUF:3 * NBUF])
    sout = list(rest[3 * NBUF:4 * NBUF])

    wid = lax.axis_index("s") * NC + lax.axis_index("c")
    base = wid * G
    pltpu.sync_copy(w_hbm, wt_v)

    def start_in(i, b):
        ug = base + i
        s = ug // 8
        bt = ug % 8
        pltpu.async_copy(x_hbm.at[s, :, bt], slabs[b], sin[b])
        pltpu.async_copy(p_hbm.at[s // 8, bt, s % 8], idxs[b], sin[b])

    def wait_in(b):
        pltpu.make_async_copy(x_hbm.at[0, :, 0], slabs[b], sin[b]).wait()
        pltpu.make_async_copy(p_hbm.at[0, 0, 0], idxs[b], sin[b]).wait()

    def start_o(i, b):
        ug = base + i
        pltpu.async_copy(slabs[b], out_hbm.at[ug // 8, :, ug % 8], sout[b])

    def wait_o(b):
        pltpu.make_async_copy(x_hbm.at[0, :, 0], slabs[b], sout[b]).wait()

    def compute(b):
        def lgroup(l, c):
            sl = pl.ds(l * 16, 16)
            rbase = idxs[b][sl] * D

            @plsc.parallel_loop(0, D, unroll=16)
            def _(e):
                v = plsc.load_gather(wt_v, [rbase + e])
                plsc.addupdate(slabs[b].at[e // 8, e % 8, sl], v)

            return c

        lax.fori_loop(0, 8, lgroup, 0)

    def outer(i2, carry):
        for u in range(NBUF):
            i = i2 * NBUF + u

            @pl.when(i < G)
            def _():
                @pl.when(i >= NBUF)
                def _():
                    wait_o(u)
                start_in(i, u)

            ib = i - 1
            bb = (u - 1) % NBUF

            @pl.when(jnp.logical_and(ib >= 0, ib < G))
            def _():
                wait_in(bb)
                compute(bb)
                start_o(ib, bb)

        return carry

    lax.fori_loop(0, STEPS // NBUF, outer, 0)

    for b in range(NBUF):
        wait_o(b)


def kernel(x, pos_seq, position_weight):
    xv = (x.transpose(1, 2, 0)
          .reshape(SEQ, 8, 8, 8, 128)
          .transpose(0, 1, 3, 2, 4))
    pv = (pos_seq.T
          .reshape(SEQ // 8, 8, 8, 128)
          .transpose(0, 2, 1, 3))
    wf = position_weight.reshape(WROWS)
    o5 = _emb_add(xv, pv, wf)
    return (o5.transpose(0, 1, 3, 2, 4)
            .reshape(SEQ, D, BATCH)
            .transpose(2, 0, 1))
